# baseline (speedup 1.0000x reference)
import functools

import jax
import jax.numpy as jnp
from jax.experimental import pallas as pl
from jax.experimental.pallas import tpu as pltpu

BETA = 1.0
MARGIN = 0.5

BLK_B = 256
T_PAD = 64


def _half_loss(x, t, m, inv_b):
    blk_b, n = x.shape
    w = pltpu.bitcast(m.reshape(blk_b * T_PAD, n), jnp.int32)
    w = w.reshape(blk_b, T_PAD // 8, n)
    w = w[:, :4, :] | w[:, 4:, :]
    w = w[:, :2, :] | w[:, 2:, :]
    words = w[:, 0, :] | w[:, 1, :]
    valid = words != 0

    lane = jax.lax.broadcasted_iota(jnp.int32, (blk_b, n), 1)
    is_t = lane == t
    neg = jnp.where(valid & (~is_t), x, jnp.float32(-10000.0))
    neg_max = jnp.max(neg, axis=1)
    pos = jnp.max(jnp.where(is_t, x, jnp.float32(-jnp.inf)), axis=1)

    z = BETA * (neg_max - pos + MARGIN)
    per = jnp.maximum(z, 0.0) + jnp.log1p(jnp.exp(-jnp.abs(z)))
    return jnp.sum(per) * inv_b


def _loss_kernel(xlo_ref, xhi_ref, tlo_ref, thi_ref, mlo_ref, mhi_ref,
                 out_ref, *, inv_b):
    part = _half_loss(xlo_ref[...], tlo_ref[...], mlo_ref[...], inv_b)
    part += _half_loss(xhi_ref[...], thi_ref[...], mhi_ref[...], inv_b)

    @pl.when(pl.program_id(0) == 0)
    def _():
        out_ref[0, 0] = jnp.float32(0.0)

    out_ref[0, 0] += part


@jax.jit
def kernel(importance_logits, target_index, person_mask):
    b, n = importance_logits.shape
    _, t_dim, _ = person_mask.shape
    tgt = target_index.astype(jnp.int32).reshape(b, 1)
    h = b // 2

    def prep(mask_half):
        return jnp.concatenate(
            [mask_half.astype(jnp.int4),
             jnp.zeros((h, T_PAD - t_dim, n), jnp.int4)],
            axis=1,
        )

    mask_lo = prep(person_mask[:h])
    mask_hi = prep(person_mask[h:])

    grid = (h // BLK_B,)
    row_spec = pl.BlockSpec((BLK_B, n), lambda i: (i, 0))
    tgt_spec = pl.BlockSpec((BLK_B, 1), lambda i: (i, 0))
    mask_spec = pl.BlockSpec((BLK_B, T_PAD, n), lambda i: (i, 0, 0))
    out = pl.pallas_call(
        functools.partial(_loss_kernel, inv_b=1.0 / b),
        grid=grid,
        in_specs=[row_spec, row_spec, tgt_spec, tgt_spec,
                  mask_spec, mask_spec],
        out_specs=pl.BlockSpec(
            (1, 1), lambda i: (0, 0), memory_space=pltpu.SMEM
        ),
        out_shape=jax.ShapeDtypeStruct((1, 1), jnp.float32),
    )(importance_logits[:h], importance_logits[h:],
      tgt[:h], tgt[h:], mask_lo, mask_hi)
    return out[0, 0]


# speedup vs baseline: 1.2006x; 1.2006x over previous
import functools

import jax
import jax.numpy as jnp
from jax.experimental import pallas as pl
from jax.experimental.pallas import tpu as pltpu

BETA = 1.0
MARGIN = 0.5

BLK_B = 256
T_PAD = 64


def _loss_kernel(logits_ref, target_ref, mask_ref, out_ref, *, inv_b):
    blk_b, n = logits_ref.shape
    x = logits_ref[...]
    t = target_ref[...]

    m = mask_ref[...]
    w = pltpu.bitcast(m.reshape(blk_b * T_PAD, n), jnp.int32)
    w = w.reshape(blk_b, T_PAD // 8, n)
    w = w[:, :4, :] | w[:, 4:, :]
    w = w[:, :2, :] | w[:, 2:, :]
    words = w[:, 0, :] | w[:, 1, :]
    valid = words != 0

    lane = jax.lax.broadcasted_iota(jnp.int32, (blk_b, n), 1)
    is_t = lane == t
    neg = jnp.where(valid & (~is_t), x, jnp.float32(-10000.0))
    neg_max = jnp.max(neg, axis=1)
    pos = jnp.max(jnp.where(is_t, x, jnp.float32(-jnp.inf)), axis=1)

    z = BETA * (neg_max - pos + MARGIN)
    per = jnp.maximum(z, 0.0) + jnp.log1p(jnp.exp(-jnp.abs(z)))
    part = jnp.sum(per) * inv_b

    @pl.when(pl.program_id(0) == 0)
    def _():
        out_ref[0, 0] = jnp.float32(0.0)

    out_ref[0, 0] += part


@jax.jit
def kernel(importance_logits, target_index, person_mask):
    b, n = importance_logits.shape
    _, t_dim, _ = person_mask.shape
    tgt = target_index.astype(jnp.int32).reshape(b, 1)
    mask_i4 = jnp.concatenate(
        [person_mask.astype(jnp.int4),
         jnp.zeros((b, T_PAD - t_dim, n), jnp.int4)],
        axis=1,
    )

    grid = (b // BLK_B,)
    out = pl.pallas_call(
        functools.partial(_loss_kernel, inv_b=1.0 / b),
        grid=grid,
        in_specs=[
            pl.BlockSpec((BLK_B, n), lambda i: (i, 0)),
            pl.BlockSpec((BLK_B, 1), lambda i: (i, 0)),
            pl.BlockSpec((BLK_B, T_PAD, n), lambda i: (i, 0, 0)),
        ],
        out_specs=pl.BlockSpec(
            (1, 1), lambda i: (0, 0), memory_space=pltpu.SMEM
        ),
        out_shape=jax.ShapeDtypeStruct((1, 1), jnp.float32),
    )(importance_logits, tgt, mask_i4)
    return out[0, 0]
